# baseline (speedup 1.0000x reference)
import dataclasses

import jax
import jax.numpy as jnp
from jax import lax
from jax.experimental import pallas as pl
from jax.experimental.pallas import tpu as pltpu
from jax.experimental.pallas import tpu_sc as plsc

B = 8
HW = 32
DISP = 63
NTASK = B * DISP
NWORK = 32
QCH = 7
NQC = DISP // QCH
IH = 16
ZROW = IH * HW


def _make_body(b_off, ntask, nrounds):
  def _body(x_hbm, out_hbm, g_ref, oa_ref, ob_ref, z_ref, sem_g, sem_a, sem_b,
            sem_z):
    wid = lax.axis_index("c") * 16 + lax.axis_index("s")
    iota = lax.iota(jnp.int32, 16)
    col_lo = iota
    col_hi = iota + 16
    zf = jnp.zeros((16,), jnp.float32)
    obufs = (oa_ref, ob_ref)
    osems = (sem_a, sem_b)

    for zr in range(ZROW, ZROW + HW):
        g_ref[zr, pl.ds(0, 16)] = zf
        g_ref[zr, pl.ds(16, 16)] = zf

    for zq in range(QCH):
        for zc in range(IH * 2):
            z_ref[zq, pl.ds(zc * 16, 16)] = zf

    @pl.loop(0, nrounds)
    def _rounds(rnd):
        t = wid + rnd * NWORK

        @pl.when(t < ntask)
        def _():
            b_out = t // DISP
            b = b_off + b_out
            p = t % DISP
            i0 = jnp.maximum(0, 31 - p)
            i1 = jnp.minimum(32, 63 - p)
            has = (i0 < IH, i1 > IH)

            for ih in range(2):
                iL = ih * IH

                @pl.when(jnp.logical_not(has[ih]))
                def _zero_half():
                    for qc in range(NQC):
                        pltpu.async_copy(
                            z_ref,
                            out_hbm.at[b_out, pl.ds(p * DISP + qc * QCH, QCH), ih, :],
                            sem_z,
                        )
                    for qc in range(NQC):
                        pltpu.make_async_copy(
                            z_ref,
                            out_hbm.at[b_out, pl.ds(p * DISP + qc * QCH, QCH), ih, :],
                            sem_z,
                        ).wait()

                @pl.when(has[ih])
                def _data_half():
                    z0 = jnp.clip(i0 - iL, 0, IH)
                    z1 = jnp.clip(i1 - iL, 0, IH)
                    nvh = jnp.maximum(z1 - z0, 0)

                    @pl.loop(0, nvh)
                    def _fire(irel):
                        i = iL + z0 + irel
                        r = p + i - 31
                        pltpu.async_copy(
                            x_hbm.at[b, r, :, i, :],
                            g_ref.at[pl.ds(irel * 32, 32), :],
                            sem_g,
                        )

                    @pl.loop(0, nvh)
                    def _drain(irel):
                        i = iL + z0 + irel
                        r = p + i - 31
                        pltpu.make_async_copy(
                            x_hbm.at[b, r, :, i, :],
                            g_ref.at[pl.ds(irel * 32, 32), :],
                            sem_g,
                        ).wait()

                    for qc in range(NQC):
                        k = ih * NQC + qc
                        o_ref = obufs[k % 2]
                        o_sem = osems[k % 2]

                        if k >= 2:
                            pk = k - 2
                            pih, pqc = pk // NQC, pk % NQC

                            def _dr(pih=pih, pqc=pqc, o_ref=o_ref, o_sem=o_sem):
                                pltpu.make_async_copy(
                                    o_ref,
                                    out_hbm.at[
                                        b_out,
                                        pl.ds(p * DISP + pqc * QCH, QCH),
                                        pih, :],
                                    o_sem,
                                ).wait()

                            if pk // NQC == ih:
                                _dr()
                            else:
                                pl.when(has[pk // NQC])(_dr)

                        @pl.loop(0, QCH)
                        def _chan(q):
                            qa = qc * QCH + q
                            vq_lo = col_lo + qa
                            vq_hi = col_hi + qa
                            m_lo = (vq_lo >= 31) & (vq_lo <= 62)
                            m_hi = (vq_hi >= 31) & (vq_hi <= 62)
                            crow_lo = jnp.clip(vq_lo - 31, 0, 31)
                            crow_hi = jnp.clip(vq_hi - 31, 0, 31)

                            for ii in range(IH):
                                valid = (ii >= z0) & (ii < z1)
                                sbase = jnp.where(valid, (ii - z0) * 32, ZROW)
                                row_lo = sbase + crow_lo
                                row_hi = sbase + crow_hi
                                glo = plsc.load_gather(g_ref, [row_lo, col_lo])
                                ghi = plsc.load_gather(g_ref, [row_hi, col_hi])
                                o_ref[q, pl.ds(ii * 32, 16)] = jnp.where(
                                    m_lo, glo, 0.0)
                                o_ref[q, pl.ds(ii * 32 + 16, 16)] = jnp.where(
                                    m_hi, ghi, 0.0)

                        pltpu.async_copy(
                            o_ref,
                            out_hbm.at[b_out, pl.ds(p * DISP + qc * QCH, QCH), ih, :],
                            o_sem,
                        )

            for ih in range(2):
                last = (2 * NQC - 2, 2 * NQC - 1) if ih == 1 else (NQC - 2, NQC - 1)
                cond = has[1] if ih == 1 else jnp.logical_not(has[1])

                def _tail(last=last):
                    for k in last:
                        pih, pqc = k // NQC, k % NQC
                        pltpu.make_async_copy(
                            obufs[k % 2],
                            out_hbm.at[b_out, pl.ds(p * DISP + pqc * QCH, QCH), pih, :],
                            osems[k % 2],
                        ).wait()

                pl.when(cond)(_tail)
  return _body


_W = DISP * DISP * HW * HW
_NCH = DISP * DISP


def _tc_combine(h0, h1):
    def body(a_ref, b_ref, o_ref):
        g = pl.program_id(0)

        @pl.when(g == 0)
        def _():
            o_ref[...] = a_ref[...]

        @pl.when(g == 1)
        def _():
            o_ref[...] = b_ref[...]

    return pl.pallas_call(
        body,
        grid=(2, B // 2, 8),
        in_specs=[
            pl.BlockSpec((1, _NCH, 128), lambda g, b, k: (b, 0, k)),
            pl.BlockSpec((1, _NCH, 128), lambda g, b, k: (b, 0, k)),
        ],
        out_specs=pl.BlockSpec(
            (1, _NCH, 128), lambda g, b, k: (g * (B // 2) + b, 0, k)),
        out_shape=jax.ShapeDtypeStruct((B, _NCH, 1024), jnp.float32),
    )(h0, h1)


def kernel(correlation_tensor):
    x = correlation_tensor.reshape(B, HW, HW, HW, HW)
    mesh = plsc.VectorSubcoreMesh(core_axis_name="c", subcore_axis_name="s")
    cp = pltpu.CompilerParams()
    if "needs_layout_passes" in pltpu.CompilerParams.__dataclass_fields__:
        cp = dataclasses.replace(cp, needs_layout_passes=False)
    halves = []
    NB = 4
    for b_off in range(0, B, NB):
        run = pl.kernel(
            _make_body(b_off, NB * DISP, (NB * DISP + NWORK - 1) // NWORK),
            out_type=jax.ShapeDtypeStruct((NB, DISP * DISP, 2, IH * HW), jnp.float32),
            mesh=mesh,
            scratch_types=[
                pltpu.VMEM((IH * HW + HW, HW), jnp.float32),
                pltpu.VMEM((QCH, IH * HW), jnp.float32),
                pltpu.VMEM((QCH, IH * HW), jnp.float32),
                pltpu.VMEM((QCH, IH * HW), jnp.float32),
                pltpu.SemaphoreType.DMA,
                pltpu.SemaphoreType.DMA,
                pltpu.SemaphoreType.DMA,
                pltpu.SemaphoreType.DMA,
            ],
            compiler_params=cp,
            name=f"corr_align_sc_b{b_off}",
        )
        halves.append(run(x))
    out = jnp.concatenate(halves, axis=0)
    return out.reshape(B, DISP * DISP, HW, HW)


# speedup vs baseline: 1.0203x; 1.0203x over previous
import dataclasses

import jax
import jax.numpy as jnp
from jax import lax
from jax.experimental import pallas as pl
from jax.experimental.pallas import tpu as pltpu
from jax.experimental.pallas import tpu_sc as plsc

B = 8
HW = 32
DISP = 63
NTASK = B * DISP
NWORK = 32
QCH = 9
NQC = DISP // QCH
IH = 16
ZROW = IH * HW


def _make_body(b_off, ntask, nrounds):
  def _body(x_hbm, out_hbm, g_ref, oa_ref, ob_ref, z_ref, sem_g, sem_a, sem_b,
            sem_z):
    wid = lax.axis_index("c") * 16 + lax.axis_index("s")
    iota = lax.iota(jnp.int32, 16)
    col_lo = iota
    col_hi = iota + 16
    zf = jnp.zeros((16,), jnp.float32)
    obufs = (oa_ref, ob_ref)
    osems = (sem_a, sem_b)

    for zr in range(ZROW, ZROW + HW):
        g_ref[zr, pl.ds(0, 16)] = zf
        g_ref[zr, pl.ds(16, 16)] = zf

    for zq in range(QCH):
        for zc in range(IH * 2):
            z_ref[zq, pl.ds(zc * 16, 16)] = zf

    @pl.loop(0, nrounds)
    def _rounds(rnd):
        t = wid + rnd * NWORK

        @pl.when(t < ntask)
        def _():
            b_out = t // DISP
            b = b_off + b_out
            p = t % DISP
            i0 = jnp.maximum(0, 31 - p)
            i1 = jnp.minimum(32, 63 - p)
            has = (i0 < IH, i1 > IH)

            for ih in range(2):
                iL = ih * IH

                @pl.when(jnp.logical_not(has[ih]))
                def _zero_half():
                    for qc in range(NQC):
                        pltpu.async_copy(
                            z_ref,
                            out_hbm.at[b_out, pl.ds(p * DISP + qc * QCH, QCH), ih, :],
                            sem_z,
                        )
                    for qc in range(NQC):
                        pltpu.make_async_copy(
                            z_ref,
                            out_hbm.at[b_out, pl.ds(p * DISP + qc * QCH, QCH), ih, :],
                            sem_z,
                        ).wait()

                @pl.when(has[ih])
                def _data_half():
                    z0 = jnp.clip(i0 - iL, 0, IH)
                    z1 = jnp.clip(i1 - iL, 0, IH)
                    nvh = jnp.maximum(z1 - z0, 0)

                    @pl.loop(0, nvh)
                    def _fire(irel):
                        i = iL + z0 + irel
                        r = p + i - 31
                        pltpu.async_copy(
                            x_hbm.at[b, r, :, i, :],
                            g_ref.at[pl.ds(irel * 32, 32), :],
                            sem_g,
                        )

                    @pl.loop(0, nvh)
                    def _drain(irel):
                        i = iL + z0 + irel
                        r = p + i - 31
                        pltpu.make_async_copy(
                            x_hbm.at[b, r, :, i, :],
                            g_ref.at[pl.ds(irel * 32, 32), :],
                            sem_g,
                        ).wait()

                    for qc in range(NQC):
                        k = ih * NQC + qc
                        o_ref = obufs[k % 2]
                        o_sem = osems[k % 2]

                        if k >= 2:
                            pk = k - 2
                            pih, pqc = pk // NQC, pk % NQC

                            def _dr(pih=pih, pqc=pqc, o_ref=o_ref, o_sem=o_sem):
                                pltpu.make_async_copy(
                                    o_ref,
                                    out_hbm.at[
                                        b_out,
                                        pl.ds(p * DISP + pqc * QCH, QCH),
                                        pih, :],
                                    o_sem,
                                ).wait()

                            if pk // NQC == ih:
                                _dr()
                            else:
                                pl.when(has[pk // NQC])(_dr)

                        @pl.loop(0, QCH)
                        def _chan(q):
                            qa = qc * QCH + q
                            vq_lo = col_lo + qa
                            vq_hi = col_hi + qa
                            m_lo = (vq_lo >= 31) & (vq_lo <= 62)
                            m_hi = (vq_hi >= 31) & (vq_hi <= 62)
                            crow_lo = jnp.clip(vq_lo - 31, 0, 31)
                            crow_hi = jnp.clip(vq_hi - 31, 0, 31)

                            for ii in range(IH):
                                valid = (ii >= z0) & (ii < z1)
                                sbase = jnp.where(valid, (ii - z0) * 32, ZROW)
                                row_lo = sbase + crow_lo
                                row_hi = sbase + crow_hi
                                glo = plsc.load_gather(g_ref, [row_lo, col_lo])
                                ghi = plsc.load_gather(g_ref, [row_hi, col_hi])
                                o_ref[q, pl.ds(ii * 32, 16)] = jnp.where(
                                    m_lo, glo, 0.0)
                                o_ref[q, pl.ds(ii * 32 + 16, 16)] = jnp.where(
                                    m_hi, ghi, 0.0)

                        pltpu.async_copy(
                            o_ref,
                            out_hbm.at[b_out, pl.ds(p * DISP + qc * QCH, QCH), ih, :],
                            o_sem,
                        )

            for ih in range(2):
                last = (2 * NQC - 2, 2 * NQC - 1) if ih == 1 else (NQC - 2, NQC - 1)
                cond = has[1] if ih == 1 else jnp.logical_not(has[1])

                def _tail(last=last):
                    for k in last:
                        pih, pqc = k // NQC, k % NQC
                        pltpu.make_async_copy(
                            obufs[k % 2],
                            out_hbm.at[b_out, pl.ds(p * DISP + pqc * QCH, QCH), pih, :],
                            osems[k % 2],
                        ).wait()

                pl.when(cond)(_tail)
  return _body


_W = DISP * DISP * HW * HW
_NCH = DISP * DISP


def _tc_combine(h0, h1):
    def body(a_ref, b_ref, o_ref):
        g = pl.program_id(0)

        @pl.when(g == 0)
        def _():
            o_ref[...] = a_ref[...]

        @pl.when(g == 1)
        def _():
            o_ref[...] = b_ref[...]

    return pl.pallas_call(
        body,
        grid=(2, B // 2, 8),
        in_specs=[
            pl.BlockSpec((1, _NCH, 128), lambda g, b, k: (b, 0, k)),
            pl.BlockSpec((1, _NCH, 128), lambda g, b, k: (b, 0, k)),
        ],
        out_specs=pl.BlockSpec(
            (1, _NCH, 128), lambda g, b, k: (g * (B // 2) + b, 0, k)),
        out_shape=jax.ShapeDtypeStruct((B, _NCH, 1024), jnp.float32),
    )(h0, h1)


def kernel(correlation_tensor):
    x = correlation_tensor.reshape(B, HW, HW, HW, HW)
    mesh = plsc.VectorSubcoreMesh(core_axis_name="c", subcore_axis_name="s")
    cp = pltpu.CompilerParams()
    if "needs_layout_passes" in pltpu.CompilerParams.__dataclass_fields__:
        cp = dataclasses.replace(cp, needs_layout_passes=False)
    halves = []
    NB = 4
    for b_off in range(0, B, NB):
        run = pl.kernel(
            _make_body(b_off, NB * DISP, (NB * DISP + NWORK - 1) // NWORK),
            out_type=jax.ShapeDtypeStruct((NB, DISP * DISP, 2, IH * HW), jnp.float32),
            mesh=mesh,
            scratch_types=[
                pltpu.VMEM((IH * HW + HW, HW), jnp.float32),
                pltpu.VMEM((QCH, IH * HW), jnp.float32),
                pltpu.VMEM((QCH, IH * HW), jnp.float32),
                pltpu.VMEM((QCH, IH * HW), jnp.float32),
                pltpu.SemaphoreType.DMA,
                pltpu.SemaphoreType.DMA,
                pltpu.SemaphoreType.DMA,
                pltpu.SemaphoreType.DMA,
            ],
            compiler_params=cp,
            name=f"corr_align_sc_b{b_off}",
        )
        halves.append(run(x))
    out = jnp.concatenate(halves, axis=0)
    return out.reshape(B, DISP * DISP, HW, HW)
